# baseline (speedup 1.0000x reference)
import functools

import jax
import jax.numpy as jnp
from jax import lax
from jax.experimental import pallas as pl
from jax.experimental.pallas import tpu as pltpu
from jax.experimental.pallas import tpu_sc as plsc

NUM_NODES = 50000
NUM_REL = 8
HID = 64
NUM_EDGES = 800000

NSC = 2
NTEC = 16
H0 = 25088
H1 = NUM_NODES - H0
SROWS = 50048
F0 = H0 // NTEC
F1 = 1560
ACC_ROWS = 25216
TRASH_ROW = 25200
CNT_STRIDE = H0
CNT_PER = 202752
CNT_TRASH = 202000
EPAD = 819200
EPT = EPAD // NTEC
CHUNK = 2048
NCHUNK = EPT // CHUNK
GROUPS = CHUNK // 16
SCHUNK = 6400
FB = 128
FIRE_AT = FB - 16

_BLK = 2000

_MESH = plsc.VectorSubcoreMesh(core_axis_name="c", subcore_axis_name="s")


def _dense_body(do_relu, h_ref, s_ref, cnt_ref, wroot_ref, wrel_ref, b_ref, o_ref):
    acc = jnp.dot(h_ref[...], wroot_ref[...],
                  preferred_element_type=jnp.float32) + b_ref[...][None, :]
    for r in range(NUM_REL):
        inv_r = 1.0 / jnp.maximum(cnt_ref[:, r], 1.0)
        sr = s_ref[r] * inv_r[:, None]
        acc += jnp.dot(sr, wrel_ref[r], preferred_element_type=jnp.float32)
    if do_relu:
        acc = jnp.maximum(acc, 0.0)
    o_ref[...] = acc


def _dense_stage(h, S, cntT, W_root, W_rel, b, do_relu):
    n = h.shape[0]
    grid = (n // _BLK,)
    return pl.pallas_call(
        functools.partial(_dense_body, do_relu),
        grid=grid,
        in_specs=[
            pl.BlockSpec((_BLK, HID), lambda i: (i, 0)),
            pl.BlockSpec((NUM_REL, _BLK, HID), lambda i: (0, i, 0)),
            pl.BlockSpec((_BLK, NUM_REL), lambda i: (i, 0)),
            pl.BlockSpec((HID, HID), lambda i: (0, 0)),
            pl.BlockSpec((NUM_REL, HID, HID), lambda i: (0, 0, 0)),
            pl.BlockSpec((HID,), lambda i: (0,)),
        ],
        out_specs=pl.BlockSpec((_BLK, HID), lambda i: (i, 0)),
        out_shape=jax.ShapeDtypeStruct((n, HID), jnp.float32),
    )(h, S, cntT, W_root, W_rel, b)


@functools.partial(
    pl.kernel, mesh=_MESH,
    compiler_params=pltpu.CompilerParams(needs_layout_passes=False),
    out_type=jax.ShapeDtypeStruct((NSC, CNT_PER), jnp.float32),
    scratch_types=[
        pltpu.VMEM((CHUNK,), jnp.int32),
        pltpu.VMEM((FB,), jnp.int32),
        pltpu.VMEM((FB,), jnp.float32),
        pltpu.VMEM((2048,), jnp.float32),
        pltpu.VMEM_SHARED((CNT_PER,), jnp.float32),
    ],
)
def _sc_count(pk_hbm, z1_hbm, cnt_out, pk_s, cidx, ones, zbuf, cnt_acc):
    c = lax.axis_index("c")
    t = lax.axis_index("s")
    lo = c * H0
    hi = H0 + c * H1

    pltpu.sync_copy(z1_hbm, zbuf)
    for j in range(FB // 16):
        ones[pl.ds(j * 16, 16)] = jnp.ones((16,), jnp.float32)
    cbase = t * (CNT_PER // NTEC)
    for k in range(6):
        pltpu.sync_copy(zbuf, cnt_acc.at[pl.ds(cbase + k * 2048, 2048)])
    pltpu.sync_copy(zbuf.at[pl.ds(0, 384)],
                    cnt_acc.at[pl.ds(cbase + 6 * 2048, 384)])

    def prefill():
        for j in range(FB // 16):
            cidx[pl.ds(j * 16, 16)] = jnp.full((16,), CNT_TRASH, jnp.int32)

    prefill()
    plsc.subcore_barrier()

    def fire():
        pltpu.sync_copy(ones, cnt_acc.at[cidx], add=True)
        prefill()

    def group_body(gi, npend):
        off = gi * 16
        pk = pk_s[pl.ds(off, 16)]
        et = lax.shift_right_logical(pk, 16)
        dv = lax.bitwise_and(pk, 0xFFFF)
        m = (et < NUM_REL) & (dv >= lo) & (dv < hi)
        ci = et * CNT_STRIDE + (dv - lo)
        cs = plsc.cumsum(jnp.where(m, 1, 0))
        pos = npend + cs - 1
        plsc.store_scatter(cidx, [pos], ci, mask=m)
        np2 = npend + cs[15]
        pl.when(np2 > FIRE_AT)(fire)
        return jnp.where(np2 > FIRE_AT, 0, np2)

    def chunk_body(ck, npend):
        pltpu.sync_copy(pk_hbm.at[pl.ds(t * EPT + ck * CHUNK, CHUNK)], pk_s)
        return lax.fori_loop(0, GROUPS, group_body, npend)

    lax.fori_loop(0, NCHUNK, chunk_body, jnp.int32(0))
    fire()
    plsc.subcore_barrier()
    pltpu.sync_copy(cnt_acc.at[pl.ds(cbase, CNT_PER // NTEC)],
                    cnt_out.at[c, pl.ds(cbase, CNT_PER // NTEC)])


@functools.partial(
    pl.kernel, mesh=_MESH,
    compiler_params=pltpu.CompilerParams(
        needs_layout_passes=False, use_tc_tiling_on_sc=False),
    out_type=jax.ShapeDtypeStruct((NUM_REL, SROWS, HID), jnp.float32),
    scratch_types=[
        pltpu.VMEM((SCHUNK,), jnp.int32),
        pltpu.VMEM((SCHUNK,), jnp.int32),
        pltpu.VMEM((FB,), jnp.int32),
        pltpu.VMEM((FB,), jnp.int32),
        pltpu.VMEM((FB,), jnp.int32),
        pltpu.VMEM((FB, HID), jnp.float32),
        pltpu.VMEM((FB, HID), jnp.float32),
        pltpu.VMEM_SHARED((ACC_ROWS, HID), jnp.float32),
        pltpu.SemaphoreType.DMA,
        pltpu.SemaphoreType.DMA,
        pltpu.SemaphoreType.DMA,
    ],
)
def _sc_scatter(h_hbm, src_hbm, pk_hbm, z2_hbm, S_out,
                pk_s, src_s, gsrc, ldst, ldst_dma, rows, zrows, acc,
                sem, semz, sems):
    c = lax.axis_index("c")
    t = lax.axis_index("s")
    lo = c * H0
    hi = H0 + c * H1

    pltpu.sync_copy(z2_hbm, zrows)

    def prefill():
        for j in range(FB // 16):
            gsrc[pl.ds(j * 16, 16)] = jnp.zeros((16,), jnp.int32)
            ldst[pl.ds(j * 16, 16)] = jnp.full((16,), TRASH_ROW, jnp.int32)

    for r in range(NUM_REL):
        zb = t * (ACC_ROWS // NTEC)
        zh = [pltpu.async_copy(zrows, acc.at[pl.ds(zb + k * 128, 128)], semz)
              for k in range(12)]
        zh.append(pltpu.async_copy(zrows.at[pl.ds(0, 40)],
                                   acc.at[pl.ds(zb + 12 * 128, 40)], semz))
        for hnd in zh:
            hnd.wait()
        prefill()
        plsc.subcore_barrier()

        def wait_prev_scatter():
            pltpu.make_async_copy(rows, acc.at[ldst_dma], sems).wait()

        def fire(fcnt):
            pl.when(fcnt > 0)(wait_prev_scatter)
            pltpu.async_copy(h_hbm.at[gsrc], rows, sem).wait()
            for j in range(FB // 16):
                ldst_dma[pl.ds(j * 16, 16)] = ldst[pl.ds(j * 16, 16)]
            pltpu.async_copy(rows, acc.at[ldst_dma], sems, add=True)
            prefill()

        def group_body(gi, carry):
            npend, fcnt = carry
            off = gi * 16
            pk = pk_s[pl.ds(off, 16)]
            sv = src_s[pl.ds(off, 16)]
            et = lax.shift_right_logical(pk, 16)
            dv = lax.bitwise_and(pk, 0xFFFF)
            m = (et == r) & (dv >= lo) & (dv < hi)
            cs = plsc.cumsum(jnp.where(m, 1, 0))
            pos = npend + cs - 1
            plsc.store_scatter(gsrc, [pos], sv, mask=m)
            plsc.store_scatter(ldst, [pos], dv - lo, mask=m)
            np2 = npend + cs[15]
            fired = np2 > FIRE_AT
            pl.when(fired)(lambda: fire(fcnt))
            return (jnp.where(fired, 0, np2),
                    jnp.where(fired, fcnt + 1, fcnt))

        def chunk_body(ck, carry):
            base = t * EPT + ck * SCHUNK
            pltpu.sync_copy(pk_hbm.at[pl.ds(base, SCHUNK)], pk_s)
            pltpu.sync_copy(src_hbm.at[pl.ds(base, SCHUNK)], src_s)
            return lax.fori_loop(0, SCHUNK // 16, group_body, carry)

        _, fcnt = lax.fori_loop(0, EPT // SCHUNK, chunk_body,
                                (jnp.int32(0), jnp.int32(0)))
        fire(fcnt)
        wait_prev_scatter()
        plsc.subcore_barrier()

        def flush0():
            b0 = t * F0
            pltpu.sync_copy(acc.at[pl.ds(b0, F0)],
                            S_out.at[r, pl.ds(b0, F0)])

        def flush1():
            b1 = t * F1
            pltpu.sync_copy(acc.at[pl.ds(b1, F1)],
                            S_out.at[r, pl.ds(H0 + b1, F1)])

        pl.when(c == 0)(flush0)
        pl.when(c == 1)(flush1)
        plsc.subcore_barrier()


def kernel(x, edge_index, edge_type, emb, W_rel1, W_root1, b1, W_rel2, W_root2, b2):
    h = jnp.take(emb, x, axis=0)
    src = edge_index[0]
    dst = edge_index[1]
    packed = jnp.left_shift(edge_type, 16) | dst
    pad = EPAD - packed.shape[0]
    packed = jnp.concatenate([packed, jnp.full((pad,), 15 << 16, jnp.int32)])
    src = jnp.concatenate([src, jnp.zeros((pad,), jnp.int32)])
    z1 = jnp.zeros((2048,), jnp.float32)
    z2 = jnp.zeros((FB, HID), jnp.float32)

    cnt_dump = _sc_count(packed, z1)
    cnt = cnt_dump[:, :NUM_REL * CNT_STRIDE].reshape(NSC, NUM_REL, CNT_STRIDE)
    cntT = jnp.concatenate([cnt[0], cnt[1, :, :H1]], axis=1).T

    S1 = _sc_scatter(h, src, packed, z2)
    h = _dense_stage(h, S1, cntT, W_root1, W_rel1, b1, True)
    S2 = _sc_scatter(h, src, packed, z2)
    return _dense_stage(h, S2, cntT, W_root2, W_rel2, b2, False)


# speedup vs baseline: 1.0089x; 1.0089x over previous
import functools

import jax
import jax.numpy as jnp
from jax import lax
from jax.experimental import pallas as pl
from jax.experimental.pallas import tpu as pltpu
from jax.experimental.pallas import tpu_sc as plsc

NUM_NODES = 50000
NUM_REL = 8
HID = 64
NUM_EDGES = 800000

NSC = 2
NTEC = 16
H0 = 25088
H1 = NUM_NODES - H0
SROWS = 50048
F0 = H0 // NTEC
F1 = 1560
ACC_ROWS = 25216
TRASH_ROW = 25200
CNT_STRIDE = H0
CNT_PER = 202752
CNT_TRASH = 202000
EPAD = 819200
EPT = EPAD // NTEC
CHUNK = 2048
NCHUNK = EPT // CHUNK
GROUPS = CHUNK // 16
SCHUNK = 6400
FB = 128
FIRE_AT = FB - 16

_BLK = 2000

_MESH = plsc.VectorSubcoreMesh(core_axis_name="c", subcore_axis_name="s")


def _dense_body(do_relu, h_ref, s_ref, cnt_ref, wroot_ref, wrel_ref, b_ref, o_ref):
    acc = jnp.dot(h_ref[...], wroot_ref[...],
                  preferred_element_type=jnp.float32) + b_ref[...][None, :]
    for r in range(NUM_REL):
        inv_r = 1.0 / jnp.maximum(cnt_ref[:, r], 1.0)
        sr = s_ref[r] * inv_r[:, None]
        acc += jnp.dot(sr, wrel_ref[r], preferred_element_type=jnp.float32)
    if do_relu:
        acc = jnp.maximum(acc, 0.0)
    o_ref[...] = acc


def _dense_stage(h, S, cntT, W_root, W_rel, b, do_relu):
    n = h.shape[0]
    grid = (n // _BLK,)
    return pl.pallas_call(
        functools.partial(_dense_body, do_relu),
        grid=grid,
        in_specs=[
            pl.BlockSpec((_BLK, HID), lambda i: (i, 0)),
            pl.BlockSpec((NUM_REL, _BLK, HID), lambda i: (0, i, 0)),
            pl.BlockSpec((_BLK, NUM_REL), lambda i: (i, 0)),
            pl.BlockSpec((HID, HID), lambda i: (0, 0)),
            pl.BlockSpec((NUM_REL, HID, HID), lambda i: (0, 0, 0)),
            pl.BlockSpec((HID,), lambda i: (0,)),
        ],
        out_specs=pl.BlockSpec((_BLK, HID), lambda i: (i, 0)),
        out_shape=jax.ShapeDtypeStruct((n, HID), jnp.float32),
    )(h, S, cntT, W_root, W_rel, b)


@functools.partial(
    pl.kernel, mesh=_MESH,
    compiler_params=pltpu.CompilerParams(needs_layout_passes=False),
    out_type=jax.ShapeDtypeStruct((NSC, CNT_PER), jnp.float32),
    scratch_types=[
        pltpu.VMEM((CHUNK,), jnp.int32),
        pltpu.VMEM((FB,), jnp.int32),
        pltpu.VMEM((FB,), jnp.float32),
        pltpu.VMEM((2048,), jnp.float32),
        pltpu.VMEM_SHARED((CNT_PER,), jnp.float32),
    ],
)
def _sc_count(pk_hbm, z1_hbm, cnt_out, pk_s, cidx, ones, zbuf, cnt_acc):
    c = lax.axis_index("c")
    t = lax.axis_index("s")
    lo = c * H0
    hi = H0 + c * H1

    pltpu.sync_copy(z1_hbm, zbuf)
    for j in range(FB // 16):
        ones[pl.ds(j * 16, 16)] = jnp.ones((16,), jnp.float32)
    cbase = t * (CNT_PER // NTEC)
    for k in range(6):
        pltpu.sync_copy(zbuf, cnt_acc.at[pl.ds(cbase + k * 2048, 2048)])
    pltpu.sync_copy(zbuf.at[pl.ds(0, 384)],
                    cnt_acc.at[pl.ds(cbase + 6 * 2048, 384)])

    def prefill():
        for j in range(FB // 16):
            cidx[pl.ds(j * 16, 16)] = jnp.full((16,), CNT_TRASH, jnp.int32)

    prefill()
    plsc.subcore_barrier()

    def fire():
        pltpu.sync_copy(ones, cnt_acc.at[cidx], add=True)
        prefill()

    def group_body(gi, npend):
        off = gi * 16
        pk = pk_s[pl.ds(off, 16)]
        et = lax.shift_right_logical(pk, 16)
        dv = lax.bitwise_and(pk, 0xFFFF)
        m = (et < NUM_REL) & (dv >= lo) & (dv < hi)
        ci = et * CNT_STRIDE + (dv - lo)
        cs = plsc.cumsum(jnp.where(m, 1, 0))
        pos = npend + cs - 1
        plsc.store_scatter(cidx, [pos], ci, mask=m)
        np2 = npend + cs[15]
        pl.when(np2 > FIRE_AT)(fire)
        return jnp.where(np2 > FIRE_AT, 0, np2)

    def chunk_body(ck, npend):
        pltpu.sync_copy(pk_hbm.at[pl.ds(t * EPT + ck * CHUNK, CHUNK)], pk_s)
        return lax.fori_loop(0, GROUPS, group_body, npend)

    lax.fori_loop(0, NCHUNK, chunk_body, jnp.int32(0))
    fire()
    plsc.subcore_barrier()
    pltpu.sync_copy(cnt_acc.at[pl.ds(cbase, CNT_PER // NTEC)],
                    cnt_out.at[c, pl.ds(cbase, CNT_PER // NTEC)])


@functools.partial(
    pl.kernel, mesh=_MESH,
    compiler_params=pltpu.CompilerParams(
        needs_layout_passes=False, use_tc_tiling_on_sc=False),
    out_type=jax.ShapeDtypeStruct((NUM_REL, SROWS, HID), jnp.float32),
    scratch_types=[
        pltpu.VMEM((SCHUNK,), jnp.int32),
        pltpu.VMEM((SCHUNK,), jnp.int32),
        pltpu.VMEM((FB,), jnp.int32),
        pltpu.VMEM((FB,), jnp.int32),
        pltpu.VMEM((FB, HID), jnp.float32),
        pltpu.VMEM((FB, HID), jnp.float32),
        pltpu.VMEM_SHARED((ACC_ROWS, HID), jnp.float32),
        pltpu.SemaphoreType.DMA,
        pltpu.SemaphoreType.DMA,
    ],
)
def _sc_scatter(h_hbm, src_hbm, pk_hbm, z2_hbm, S_out,
                pk_s, src_s, gsrc, ldst, rows, zrows, acc, sem, semz):
    c = lax.axis_index("c")
    t = lax.axis_index("s")
    lo = c * H0
    hi = H0 + c * H1

    pltpu.sync_copy(z2_hbm, zrows)

    def prefill():
        for j in range(FB // 16):
            gsrc[pl.ds(j * 16, 16)] = jnp.zeros((16,), jnp.int32)
            ldst[pl.ds(j * 16, 16)] = jnp.full((16,), TRASH_ROW, jnp.int32)

    for r in range(NUM_REL):
        zb = t * (ACC_ROWS // NTEC)
        zh = [pltpu.async_copy(zrows, acc.at[pl.ds(zb + k * 128, 128)], semz)
              for k in range(12)]
        zh.append(pltpu.async_copy(zrows.at[pl.ds(0, 40)],
                                   acc.at[pl.ds(zb + 12 * 128, 40)], semz))
        for hnd in zh:
            hnd.wait()
        prefill()
        plsc.subcore_barrier()

        def fire():
            pltpu.async_copy(h_hbm.at[gsrc], rows, sem).wait()
            pltpu.sync_copy(rows, acc.at[ldst], add=True)
            prefill()

        def group_body(gi, npend):
            off = gi * 64
            halves = []
            for u in range(4):
                pk = pk_s[pl.ds(off + u * 16, 16)]
                sv = src_s[pl.ds(off + u * 16, 16)]
                et = lax.shift_right_logical(pk, 16)
                dv = lax.bitwise_and(pk, 0xFFFF)
                m = (et == r) & (dv >= lo) & (dv < hi)
                cs = plsc.cumsum(jnp.where(m, 1, 0))
                halves.append((sv, dv, m, cs))
            for sv, dv, m, cs in halves:
                pos = npend + cs - 1
                plsc.store_scatter(gsrc, [pos], sv, mask=m)
                plsc.store_scatter(ldst, [pos], dv - lo, mask=m)
                np2 = npend + cs[15]
                pl.when(np2 > FIRE_AT)(fire)
                npend = jnp.where(np2 > FIRE_AT, 0, np2)
            return npend

        def chunk_body(ck, npend):
            base = t * EPT + ck * SCHUNK
            pltpu.sync_copy(pk_hbm.at[pl.ds(base, SCHUNK)], pk_s)
            pltpu.sync_copy(src_hbm.at[pl.ds(base, SCHUNK)], src_s)
            return lax.fori_loop(0, SCHUNK // 64, group_body, npend)

        lax.fori_loop(0, EPT // SCHUNK, chunk_body, jnp.int32(0))
        fire()
        plsc.subcore_barrier()

        def flush0():
            b0 = t * F0
            pltpu.sync_copy(acc.at[pl.ds(b0, F0)],
                            S_out.at[r, pl.ds(b0, F0)])

        def flush1():
            b1 = t * F1
            pltpu.sync_copy(acc.at[pl.ds(b1, F1)],
                            S_out.at[r, pl.ds(H0 + b1, F1)])

        pl.when(c == 0)(flush0)
        pl.when(c == 1)(flush1)
        plsc.subcore_barrier()


def kernel(x, edge_index, edge_type, emb, W_rel1, W_root1, b1, W_rel2, W_root2, b2):
    h = jnp.take(emb, x, axis=0)
    src = edge_index[0]
    dst = edge_index[1]
    packed = jnp.left_shift(edge_type, 16) | dst
    pad = EPAD - packed.shape[0]
    packed = jnp.concatenate([packed, jnp.full((pad,), 15 << 16, jnp.int32)])
    src = jnp.concatenate([src, jnp.zeros((pad,), jnp.int32)])
    z1 = jnp.zeros((2048,), jnp.float32)
    z2 = jnp.zeros((FB, HID), jnp.float32)

    cnt_dump = _sc_count(packed, z1)
    cnt = cnt_dump[:, :NUM_REL * CNT_STRIDE].reshape(NSC, NUM_REL, CNT_STRIDE)
    cntT = jnp.concatenate([cnt[0], cnt[1, :, :H1]], axis=1).T

    S1 = _sc_scatter(h, src, packed, z2)
    h = _dense_stage(h, S1, cntT, W_root1, W_rel1, b1, True)
    S2 = _sc_scatter(h, src, packed, z2)
    return _dense_stage(h, S2, cntT, W_root2, W_rel2, b2, False)
